# baseline (speedup 1.0000x reference)
import functools

import jax
import jax.numpy as jnp
from jax import lax
from jax.experimental import pallas as pl
from jax.experimental.pallas import tpu as pltpu
from jax.experimental.pallas import tpu_sc as plsc

B = 2
C = 192
H = 224
W = 224
HW = H * W

NUM_CORES = 2
NUM_SUBCORES = 16
LANES = 16

CPT = C // NUM_SUBCORES
PASSES = CPT // 2
CHUNK = 896
NCHUNK = HW // CHUNK
GROUPS = CHUNK // LANES
NBUF = 4


def _prep_pallas(phi):
    def body(phi_ref, i_ref, w_ref):
        ph0 = phi_ref[0]
        ph1 = phi_ref[1]
        ii = lax.broadcasted_iota(jnp.int32, (H, W), 0).astype(jnp.float32)
        jj = lax.broadcasted_iota(jnp.int32, (H, W), 1).astype(jnp.float32)
        gi = ii + ph0
        gj = jj + ph1
        i0f = jnp.floor(gi)
        j0f = jnp.floor(gj)
        i0 = jnp.mod(i0f.astype(jnp.int32), H)
        j0 = jnp.mod(j0f.astype(jnp.int32), W)
        i_ref[0] = i0 * W + j0
        i_ref[1] = jnp.where(j0 == W - 1, 1 - W, 1)
        i_ref[2] = jnp.where(i0 == H - 1, W - HW, W)
        w_ref[0] = gi - i0f
        w_ref[1] = gj - j0f

    return pl.pallas_call(
        body,
        grid=(B,),
        in_specs=[pl.BlockSpec((None, 2, H, W), lambda b: (b, 0, 0, 0))],
        out_specs=[
            pl.BlockSpec((None, 3, H, W), lambda b: (b, 0, 0, 0)),
            pl.BlockSpec((None, 2, H, W), lambda b: (b, 0, 0, 0)),
        ],
        out_shape=[
            jax.ShapeDtypeStruct((B, 3, H, W), jnp.int32),
            jax.ShapeDtypeStruct((B, 2, H, W), jnp.float32),
        ],
    )(phi)


def _norm_pallas(acc, cnt):
    CB = 16

    def body(a_ref, c_ref, o_ref):
        total = jnp.sum(c_ref[...], axis=0, keepdims=True)
        o_ref[...] = a_ref[...] / jnp.maximum(total, 0.001)

    return pl.pallas_call(
        body,
        grid=(B, C // CB),
        in_specs=[
            pl.BlockSpec((None, CB, HW), lambda b, c: (b, c, 0)),
            pl.BlockSpec((None, NUM_SUBCORES, HW), lambda b, c: (b, 0, 0)),
        ],
        out_specs=pl.BlockSpec((None, CB, HW), lambda b, c: (b, c, 0)),
        out_shape=jax.ShapeDtypeStruct((B, C, HW), jnp.float32),
    )(acc, cnt)


def _decode_group(ibuf, wbuf, s, g):
    base = g * LANES
    idx00 = ibuf[s, 0, pl.ds(base, LANES)]
    dj = ibuf[s, 1, pl.ds(base, LANES)]
    di = ibuf[s, 2, pl.ds(base, LANES)]
    wi = wbuf[s, 0, pl.ds(base, LANES)]
    wj = wbuf[s, 1, pl.ds(base, LANES)]
    idx01 = idx00 + dj
    idx10 = idx00 + di
    idx11 = idx01 + di
    u = 1.0 - wi
    v = 1.0 - wj
    return (
        (idx00, u * v),
        (idx01, u * wj),
        (idx10, wi * v),
        (idx11, wi * wj),
    )


def _sc_push(xr, itab, wtab):
    mesh = plsc.VectorSubcoreMesh(core_axis_name="c", subcore_axis_name="s")

    def body(x_hbm, i_hbm, w_hbm, out_hbm, cnt_hbm, acc_a, acc_b, ibuf, wbuf,
             xbuf, *sems):
        cid = lax.axis_index("c")
        sid = lax.axis_index("s")
        sem_i = sems[0:NBUF]
        sem_w = sems[NBUF:2 * NBUF]
        sem_x = sems[2 * NBUF:3 * NBUF]
        sem_oa = sems[3 * NBUF]
        sem_ob = sems[3 * NBUF + 1]

        def tab_src(t):
            return i_hbm.at[cid, t], w_hbm.at[cid, t]

        def start_tab(t, s):
            si, sw = tab_src(t)
            pltpu.async_copy(si, ibuf.at[s], sem_i[s])
            pltpu.async_copy(sw, wbuf.at[s], sem_w[s])

        def wait_tab(t, s):
            si, sw = tab_src(t)
            pltpu.make_async_copy(si, ibuf.at[s], sem_i[s]).wait()
            pltpu.make_async_copy(sw, wbuf.at[s], sem_w[s]).wait()

        def chunk_loop(compute_chunk, extra_start, extra_wait):
            for t0 in range(NBUF - 1):
                start_tab(t0, t0)
                extra_start(t0, t0)

            def quad_body(tq, carry):
                for s in range(NBUF):
                    t = tq * NBUF + s
                    sn = (s + NBUF - 1) % NBUF

                    @pl.when(t + NBUF - 1 < NCHUNK)
                    def _():
                        start_tab(t + NBUF - 1, sn)
                        extra_start(t + NBUF - 1, sn)

                    wait_tab(t, s)
                    extra_wait(t, s)
                    compute_chunk(t, s)

                return carry

            lax.fori_loop(0, NCHUNK // NBUF, quad_body, 0)

        def zero_a():
            @plsc.parallel_loop(0, HW // LANES, 1, unroll=8)
            def _(g):
                acc_a[pl.ds(g * LANES, LANES)] = jnp.zeros((LANES,),
                                                           jnp.float32)

        def zero_ab():
            @plsc.parallel_loop(0, HW // LANES, 1, unroll=8)
            def _(g):
                z = jnp.zeros((LANES,), jnp.float32)
                acc_a[pl.ds(g * LANES, LANES)] = z
                acc_b[pl.ds(g * LANES, LANES)] = z

        zero_a()

        lo = (sid * 7) >> 1
        hi = ((sid + 1) * 7) >> 1

        def cnt_body(t, carry):
            pltpu.sync_copy(i_hbm.at[cid, t], ibuf.at[0])
            pltpu.sync_copy(w_hbm.at[cid, t], wbuf.at[0])

            @plsc.parallel_loop(0, GROUPS, 1, unroll=4)
            def _(g):
                for idx, wt in _decode_group(ibuf, wbuf, 0, g):
                    plsc.addupdate_scatter(acc_a, [idx], wt)

            return carry

        lax.fori_loop(lo, hi, cnt_body, 0)
        pltpu.sync_copy(acc_a, cnt_hbm.at[cid, sid])

        c_base = sid * CPT

        for p in range(PASSES):
            c0 = c_base + 2 * p

            def x_src(t):
                off = pl.multiple_of(t * CHUNK, 8)
                return x_hbm.at[cid, pl.ds(c0, 2), pl.ds(off, CHUNK)]

            def start_x(t, s):
                pltpu.async_copy(x_src(t), xbuf.at[s], sem_x[s])

            def wait_x(t, s):
                pltpu.make_async_copy(x_src(t), xbuf.at[s], sem_x[s]).wait()

            if p > 0:
                cprev = c_base + 2 * (p - 1)
                pltpu.make_async_copy(acc_a, out_hbm.at[cid, cprev],
                                      sem_oa).wait()
                pltpu.make_async_copy(acc_b, out_hbm.at[cid, cprev + 1],
                                      sem_ob).wait()
            zero_ab()

            def scat_chunk(t, s):
                del t

                @plsc.parallel_loop(0, GROUPS, 1, unroll=4)
                def _(g):
                    base = g * LANES
                    x0 = xbuf[s, 0, pl.ds(base, LANES)]
                    x1 = xbuf[s, 1, pl.ds(base, LANES)]
                    for idx, wt in _decode_group(ibuf, wbuf, s, g):
                        plsc.addupdate_scatter(acc_a, [idx], wt * x0)
                        plsc.addupdate_scatter(acc_b, [idx], wt * x1)

            chunk_loop(scat_chunk, start_x, wait_x)

            pltpu.async_copy(acc_a, out_hbm.at[cid, c0], sem_oa)
            pltpu.async_copy(acc_b, out_hbm.at[cid, c0 + 1], sem_ob)

        clast = c_base + 2 * (PASSES - 1)
        pltpu.make_async_copy(acc_a, out_hbm.at[cid, clast], sem_oa).wait()
        pltpu.make_async_copy(acc_b, out_hbm.at[cid, clast + 1],
                              sem_ob).wait()

    f = pl.kernel(
        body,
        out_type=[
            jax.ShapeDtypeStruct((B, C, HW), jnp.float32),
            jax.ShapeDtypeStruct((B, NUM_SUBCORES, HW), jnp.float32),
        ],
        mesh=mesh,
        scratch_types=[
            pltpu.VMEM((HW,), jnp.float32),
            pltpu.VMEM((HW,), jnp.float32),
            pltpu.VMEM((NBUF, 3, CHUNK), jnp.int32),
            pltpu.VMEM((NBUF, 2, CHUNK), jnp.float32),
            pltpu.VMEM((NBUF, 2, CHUNK), jnp.float32),
        ] + [pltpu.SemaphoreType.DMA] * (3 * NBUF + 2),
        compiler_params=pltpu.CompilerParams(needs_layout_passes=False),
    )
    return f(xr, itab, wtab)


@jax.jit
def kernel(x, phi):
    itab, wtab = _prep_pallas(phi)
    xr = x.reshape(B, C, HW)
    itab = itab.reshape(B, 3, NCHUNK, CHUNK).transpose(0, 2, 1, 3)
    wtab = wtab.reshape(B, 2, NCHUNK, CHUNK).transpose(0, 2, 1, 3)
    acc, cnt = _sc_push(xr, itab, wtab)
    out = _norm_pallas(acc, cnt)
    return out.reshape(B, C, H, W)


# speedup vs baseline: 1.0560x; 1.0560x over previous
import functools

import jax
import jax.numpy as jnp
from jax import lax
from jax.experimental import pallas as pl
from jax.experimental.pallas import tpu as pltpu
from jax.experimental.pallas import tpu_sc as plsc

B = 2
C = 192
H = 224
W = 224
HW = H * W

NUM_CORES = 2
NUM_SUBCORES = 16
LANES = 16

CPT = C // NUM_SUBCORES
PASSES = CPT // 2
CHUNK = 1792
NCHUNK = HW // CHUNK
GROUPS = CHUNK // LANES
NBUF = 4
WSCALE = 1.0 / 65536.0


def _prep_pallas(phi):
    def body(phi_ref, t_ref, w_ref):
        ph0 = phi_ref[0]
        ph1 = phi_ref[1]
        ii = lax.broadcasted_iota(jnp.int32, (H, W), 0).astype(jnp.float32)
        jj = lax.broadcasted_iota(jnp.int32, (H, W), 1).astype(jnp.float32)
        gi = ii + ph0
        gj = jj + ph1
        i0f = jnp.floor(gi)
        j0f = jnp.floor(gj)
        i0 = jnp.mod(i0f.astype(jnp.int32), H)
        j0 = jnp.mod(j0f.astype(jnp.int32), W)
        jwrap = (j0 == W - 1).astype(jnp.int32)
        iwrap = (i0 == H - 1).astype(jnp.int32)
        t_ref[...] = (i0 * W + j0) | (jwrap << 16) | (iwrap << 17)
        wq_i = (jnp.clip(gi - i0f, 0.0, 1.0) * 65536.0).astype(jnp.int32)
        wq_j = (jnp.clip(gj - j0f, 0.0, 1.0) * 65536.0).astype(jnp.int32)
        wq_i = jnp.minimum(wq_i, 65535)
        wq_j = jnp.minimum(wq_j, 65535)
        w_ref[...] = wq_i | (wq_j << 16)

    return pl.pallas_call(
        body,
        grid=(B,),
        in_specs=[pl.BlockSpec((None, 2, H, W), lambda b: (b, 0, 0, 0))],
        out_specs=[
            pl.BlockSpec((None, H, W), lambda b: (b, 0, 0)),
            pl.BlockSpec((None, H, W), lambda b: (b, 0, 0)),
        ],
        out_shape=[
            jax.ShapeDtypeStruct((B, H, W), jnp.int32),
            jax.ShapeDtypeStruct((B, H, W), jnp.int32),
        ],
    )(phi)


def _norm_pallas(acc, cnt):
    CB = 16

    def body(a_ref, c_ref, o_ref):
        total = jnp.sum(c_ref[...], axis=0, keepdims=True)
        o_ref[...] = a_ref[...] / jnp.maximum(total, 0.001)

    return pl.pallas_call(
        body,
        grid=(B, C // CB),
        in_specs=[
            pl.BlockSpec((None, CB, HW), lambda b, c: (b, c, 0)),
            pl.BlockSpec((None, NUM_SUBCORES, HW), lambda b, c: (b, 0, 0)),
        ],
        out_specs=pl.BlockSpec((None, CB, HW), lambda b, c: (b, c, 0)),
        out_shape=jax.ShapeDtypeStruct((B, C, HW), jnp.float32),
    )(acc, cnt)


def _decode_group(tbuf, wbuf, s, g):
    base = g * LANES
    tw = tbuf[s, pl.ds(base, LANES)]
    ww = wbuf[s, pl.ds(base, LANES)]
    idx00 = tw & 0xFFFF
    djb = (tw >> 16) & 1
    dib = tw >> 17
    dj = 1 - djb * W
    di = W - dib * HW
    wi = ((ww & 0xFFFF).astype(jnp.float32)) * WSCALE
    wj = (((ww >> 16) & 0xFFFF).astype(jnp.float32)) * WSCALE
    idx01 = idx00 + dj
    idx10 = idx00 + di
    idx11 = idx01 + di
    u = 1.0 - wi
    v = 1.0 - wj
    return (
        (idx00, u * v),
        (idx01, u * wj),
        (idx10, wi * v),
        (idx11, wi * wj),
    )


def _sc_push(xr, ttab, wtab):
    mesh = plsc.VectorSubcoreMesh(core_axis_name="c", subcore_axis_name="s")

    def body(x_hbm, t_hbm, w_hbm, out_hbm, cnt_hbm, acc_a, acc_b, tbuf, wbuf,
             xbuf, *sems):
        cid = lax.axis_index("c")
        sid = lax.axis_index("s")
        sem_t = sems[0:NBUF]
        sem_w = sems[NBUF:2 * NBUF]
        sem_x = sems[2 * NBUF:3 * NBUF]
        sem_oa = sems[3 * NBUF]
        sem_ob = sems[3 * NBUF + 1]

        def tab_src(t):
            return t_hbm.at[cid, t], w_hbm.at[cid, t]

        def start_tab(t, s):
            st, sw = tab_src(t)
            pltpu.async_copy(st, tbuf.at[s], sem_t[s])
            pltpu.async_copy(sw, wbuf.at[s], sem_w[s])

        def wait_tab(t, s):
            st, sw = tab_src(t)
            pltpu.make_async_copy(st, tbuf.at[s], sem_t[s]).wait()
            pltpu.make_async_copy(sw, wbuf.at[s], sem_w[s]).wait()

        def chunk_loop(compute_chunk, extra_start, extra_wait):
            for t0 in range(NBUF - 1):
                start_tab(t0, t0)
                extra_start(t0, t0)

            def quad_body(tq, carry):
                for s in range(NBUF):
                    t = tq * NBUF + s
                    sn = (s + NBUF - 1) % NBUF

                    @pl.when(t + NBUF - 1 < NCHUNK)
                    def _():
                        start_tab(t + NBUF - 1, sn)
                        extra_start(t + NBUF - 1, sn)

                    wait_tab(t, s)
                    extra_wait(t, s)
                    compute_chunk(t, s)

                return carry

            lax.fori_loop(0, NCHUNK // NBUF, quad_body, 0)

        def zero_a():
            @plsc.parallel_loop(0, HW // LANES, 1, unroll=8)
            def _(g):
                acc_a[pl.ds(g * LANES, LANES)] = jnp.zeros((LANES,),
                                                           jnp.float32)

        def zero_ab():
            @plsc.parallel_loop(0, HW // LANES, 1, unroll=8)
            def _(g):
                z = jnp.zeros((LANES,), jnp.float32)
                acc_a[pl.ds(g * LANES, LANES)] = z
                acc_b[pl.ds(g * LANES, LANES)] = z

        zero_a()

        lo = (sid * 7) >> 2
        hi = ((sid + 1) * 7) >> 2

        def cnt_body(t, carry):
            pltpu.sync_copy(t_hbm.at[cid, t], tbuf.at[0])
            pltpu.sync_copy(w_hbm.at[cid, t], wbuf.at[0])

            @plsc.parallel_loop(0, GROUPS, 1, unroll=4)
            def _(g):
                for idx, wt in _decode_group(tbuf, wbuf, 0, g):
                    plsc.addupdate_scatter(acc_a, [idx], wt)

            return carry

        lax.fori_loop(lo, hi, cnt_body, 0)
        pltpu.sync_copy(acc_a, cnt_hbm.at[cid, sid])

        c_base = sid * CPT

        for p in range(PASSES):
            c0 = c_base + 2 * p

            def x_src(t):
                off = pl.multiple_of(t * CHUNK, 8)
                return x_hbm.at[cid, pl.ds(c0, 2), pl.ds(off, CHUNK)]

            def start_x(t, s):
                pltpu.async_copy(x_src(t), xbuf.at[s], sem_x[s])

            def wait_x(t, s):
                pltpu.make_async_copy(x_src(t), xbuf.at[s], sem_x[s]).wait()

            if p > 0:
                cprev = c_base + 2 * (p - 1)
                pltpu.make_async_copy(acc_a, out_hbm.at[cid, cprev],
                                      sem_oa).wait()
                pltpu.make_async_copy(acc_b, out_hbm.at[cid, cprev + 1],
                                      sem_ob).wait()
            zero_ab()

            def scat_chunk(t, s):
                del t

                @plsc.parallel_loop(0, GROUPS, 1, unroll=4)
                def _(g):
                    base = g * LANES
                    x0 = xbuf[s, 0, pl.ds(base, LANES)]
                    x1 = xbuf[s, 1, pl.ds(base, LANES)]
                    for idx, wt in _decode_group(tbuf, wbuf, s, g):
                        plsc.addupdate_scatter(acc_a, [idx], wt * x0)
                        plsc.addupdate_scatter(acc_b, [idx], wt * x1)

            chunk_loop(scat_chunk, start_x, wait_x)

            pltpu.async_copy(acc_a, out_hbm.at[cid, c0], sem_oa)
            pltpu.async_copy(acc_b, out_hbm.at[cid, c0 + 1], sem_ob)

        clast = c_base + 2 * (PASSES - 1)
        pltpu.make_async_copy(acc_a, out_hbm.at[cid, clast], sem_oa).wait()
        pltpu.make_async_copy(acc_b, out_hbm.at[cid, clast + 1],
                              sem_ob).wait()

    f = pl.kernel(
        body,
        out_type=[
            jax.ShapeDtypeStruct((B, C, HW), jnp.float32),
            jax.ShapeDtypeStruct((B, NUM_SUBCORES, HW), jnp.float32),
        ],
        mesh=mesh,
        scratch_types=[
            pltpu.VMEM((HW,), jnp.float32),
            pltpu.VMEM((HW,), jnp.float32),
            pltpu.VMEM((NBUF, CHUNK), jnp.int32),
            pltpu.VMEM((NBUF, CHUNK), jnp.int32),
            pltpu.VMEM((NBUF, 2, CHUNK), jnp.float32),
        ] + [pltpu.SemaphoreType.DMA] * (3 * NBUF + 2),
        compiler_params=pltpu.CompilerParams(needs_layout_passes=False),
    )
    return f(xr, ttab, wtab)


@jax.jit
def kernel(x, phi):
    ttab, wtab = _prep_pallas(phi)
    xr = x.reshape(B, C, HW)
    acc, cnt = _sc_push(xr, ttab.reshape(B, NCHUNK, CHUNK),
                        wtab.reshape(B, NCHUNK, CHUNK))
    out = _norm_pallas(acc, cnt)
    return out.reshape(B, C, H, W)


# speedup vs baseline: 1.2104x; 1.1463x over previous
import functools

import jax
import jax.numpy as jnp
from jax import lax
from jax.experimental import pallas as pl
from jax.experimental.pallas import tpu as pltpu
from jax.experimental.pallas import tpu_sc as plsc

B = 2
C = 192
H = 224
W = 224
HW = H * W

NUM_CORES = 2
NUM_SUBCORES = 16
LANES = 16

CPT = C // NUM_SUBCORES
PASSES = CPT // 2
CHUNK = 1792
NCHUNK = HW // CHUNK
GROUPS = CHUNK // LANES
NBUF = 4
WSCALE = 1.0 / 65536.0


def _prep_pallas(phi):
    def body(phi_ref, t_ref, w_ref):
        ph0 = phi_ref[0]
        ph1 = phi_ref[1]
        ii = lax.broadcasted_iota(jnp.int32, (H, W), 0).astype(jnp.float32)
        jj = lax.broadcasted_iota(jnp.int32, (H, W), 1).astype(jnp.float32)
        gi = ii + ph0
        gj = jj + ph1
        i0f = jnp.floor(gi)
        j0f = jnp.floor(gj)
        i0 = jnp.mod(i0f.astype(jnp.int32), H)
        j0 = jnp.mod(j0f.astype(jnp.int32), W)
        jwrap = (j0 == W - 1).astype(jnp.int32)
        iwrap = (i0 == H - 1).astype(jnp.int32)
        t_ref[...] = (i0 * W + j0) | (jwrap << 16) | (iwrap << 17)
        wq_i = (jnp.clip(gi - i0f, 0.0, 1.0) * 65536.0).astype(jnp.int32)
        wq_j = (jnp.clip(gj - j0f, 0.0, 1.0) * 65536.0).astype(jnp.int32)
        wq_i = jnp.minimum(wq_i, 65535)
        wq_j = jnp.minimum(wq_j, 65535)
        w_ref[...] = wq_i | (wq_j << 16)

    return pl.pallas_call(
        body,
        grid=(B,),
        in_specs=[pl.BlockSpec((None, 2, H, W), lambda b: (b, 0, 0, 0))],
        out_specs=[
            pl.BlockSpec((None, H, W), lambda b: (b, 0, 0)),
            pl.BlockSpec((None, H, W), lambda b: (b, 0, 0)),
        ],
        out_shape=[
            jax.ShapeDtypeStruct((B, H, W), jnp.int32),
            jax.ShapeDtypeStruct((B, H, W), jnp.int32),
        ],
    )(phi)


def _norm_pallas(acc, cnt):
    CB = 16

    def body(a_ref, c_ref, o_ref):
        total = jnp.sum(c_ref[...], axis=0, keepdims=True)
        o_ref[...] = a_ref[...] / jnp.maximum(total, 0.001)

    return pl.pallas_call(
        body,
        grid=(B, C // CB),
        in_specs=[
            pl.BlockSpec((None, CB, HW), lambda b, c: (b, c, 0)),
            pl.BlockSpec((None, NUM_SUBCORES, HW), lambda b, c: (b, 0, 0)),
        ],
        out_specs=pl.BlockSpec((None, CB, HW), lambda b, c: (b, c, 0)),
        out_shape=jax.ShapeDtypeStruct((B, C, HW), jnp.float32),
    )(acc, cnt)


def _decode_group(tbuf, wbuf, s, g):
    base = g * LANES
    tw = tbuf[s, pl.ds(base, LANES)]
    ww = wbuf[s, pl.ds(base, LANES)]
    idx00 = tw & 0xFFFF
    djb = (tw >> 16) & 1
    dib = tw >> 17
    dj = 1 - djb * W
    di = W - dib * HW
    wi = ((ww & 0xFFFF).astype(jnp.float32)) * WSCALE
    wj = (((ww >> 16) & 0xFFFF).astype(jnp.float32)) * WSCALE
    idx01 = idx00 + dj
    idx10 = idx00 + di
    idx11 = idx01 + di
    u = 1.0 - wi
    v = 1.0 - wj
    return (
        (idx00, u * v),
        (idx01, u * wj),
        (idx10, wi * v),
        (idx11, wi * wj),
    )


def _sc_push(xr, ttab, wtab):
    mesh = plsc.VectorSubcoreMesh(core_axis_name="c", subcore_axis_name="s")

    def body(x_hbm, t_hbm, w_hbm, out_hbm, cnt_hbm, acc_a, acc_b, tbuf, wbuf,
             xbuf, *sems):
        cid = lax.axis_index("c")
        sid = lax.axis_index("s")
        sem_t = sems[0:NBUF]
        sem_w = sems[NBUF:2 * NBUF]
        sem_x = sems[2 * NBUF:3 * NBUF]
        sem_oa = sems[3 * NBUF]
        sem_ob = sems[3 * NBUF + 1]

        def tab_src(t):
            return t_hbm.at[cid, t], w_hbm.at[cid, t]

        def start_tab(t, s):
            st, sw = tab_src(t)
            pltpu.async_copy(st, tbuf.at[s], sem_t[s])
            pltpu.async_copy(sw, wbuf.at[s], sem_w[s])

        def wait_tab(t, s):
            st, sw = tab_src(t)
            pltpu.make_async_copy(st, tbuf.at[s], sem_t[s]).wait()
            pltpu.make_async_copy(sw, wbuf.at[s], sem_w[s]).wait()

        def chunk_loop(compute_chunk, extra_start, extra_wait):
            for t0 in range(NBUF - 1):
                start_tab(t0, t0)
                extra_start(t0, t0)

            def quad_body(tq, carry):
                for s in range(NBUF):
                    t = tq * NBUF + s
                    sn = (s + NBUF - 1) % NBUF

                    @pl.when(t + NBUF - 1 < NCHUNK)
                    def _():
                        start_tab(t + NBUF - 1, sn)
                        extra_start(t + NBUF - 1, sn)

                    wait_tab(t, s)
                    extra_wait(t, s)
                    compute_chunk(t, s)

                return carry

            lax.fori_loop(0, NCHUNK // NBUF, quad_body, 0)

        def zero_a():
            @plsc.parallel_loop(0, HW // LANES, 1, unroll=8)
            def _(g):
                acc_a[pl.ds(g * LANES, LANES)] = jnp.zeros((LANES,),
                                                           jnp.float32)

        def zero_ab():
            @plsc.parallel_loop(0, HW // LANES, 1, unroll=8)
            def _(g):
                z = jnp.zeros((LANES,), jnp.float32)
                acc_a[pl.ds(g * LANES, LANES)] = z
                acc_b[pl.ds(g * LANES, LANES)] = z

        zero_a()

        lo = (sid * 7) >> 2
        hi = ((sid + 1) * 7) >> 2

        def cnt_body(t, carry):
            pltpu.sync_copy(t_hbm.at[cid, t], tbuf.at[0])
            pltpu.sync_copy(w_hbm.at[cid, t], wbuf.at[0])

            @plsc.parallel_loop(0, GROUPS, 1, unroll=4)
            def _(g):
                for idx, wt in _decode_group(tbuf, wbuf, 0, g):
                    plsc.addupdate_scatter(acc_a, [idx], wt)

            return carry

        lax.fori_loop(lo, hi, cnt_body, 0)
        pltpu.sync_copy(acc_a, cnt_hbm.at[cid, sid])

        c_base = sid * CPT

        for p in range(PASSES):
            c0 = c_base + 2 * p

            def x_src(t):
                roff = pl.multiple_of(t * (CHUNK // W), 8)
                return x_hbm.at[cid, pl.ds(c0, 2), pl.ds(roff, CHUNK // W), :]

            def start_x(t, s):
                pltpu.async_copy(x_src(t), xbuf.at[s], sem_x[s])

            def wait_x(t, s):
                pltpu.make_async_copy(x_src(t), xbuf.at[s], sem_x[s]).wait()

            if p > 0:
                cprev = c_base + 2 * (p - 1)
                pltpu.make_async_copy(acc_a, out_hbm.at[cid, cprev],
                                      sem_oa).wait()
                pltpu.make_async_copy(acc_b, out_hbm.at[cid, cprev + 1],
                                      sem_ob).wait()
            zero_ab()

            def scat_chunk(t, s):
                del t

                @plsc.parallel_loop(0, GROUPS, 1, unroll=4)
                def _(g):
                    base = g * LANES
                    q = (g * 9363) >> 17
                    col = (g - q * 14) * LANES
                    x0 = xbuf[s, 0, q, pl.ds(col, LANES)]
                    x1 = xbuf[s, 1, q, pl.ds(col, LANES)]
                    for idx, wt in _decode_group(tbuf, wbuf, s, g):
                        plsc.addupdate_scatter(acc_a, [idx], wt * x0)
                        plsc.addupdate_scatter(acc_b, [idx], wt * x1)

            chunk_loop(scat_chunk, start_x, wait_x)

            pltpu.async_copy(acc_a, out_hbm.at[cid, c0], sem_oa)
            pltpu.async_copy(acc_b, out_hbm.at[cid, c0 + 1], sem_ob)

        clast = c_base + 2 * (PASSES - 1)
        pltpu.make_async_copy(acc_a, out_hbm.at[cid, clast], sem_oa).wait()
        pltpu.make_async_copy(acc_b, out_hbm.at[cid, clast + 1],
                              sem_ob).wait()

    f = pl.kernel(
        body,
        out_type=[
            jax.ShapeDtypeStruct((B, C, HW), jnp.float32),
            jax.ShapeDtypeStruct((B, NUM_SUBCORES, HW), jnp.float32),
        ],
        mesh=mesh,
        scratch_types=[
            pltpu.VMEM((HW,), jnp.float32),
            pltpu.VMEM((HW,), jnp.float32),
            pltpu.VMEM((NBUF, CHUNK), jnp.int32),
            pltpu.VMEM((NBUF, CHUNK), jnp.int32),
            pltpu.VMEM((NBUF, 2, CHUNK // W, W), jnp.float32),
        ] + [pltpu.SemaphoreType.DMA] * (3 * NBUF + 2),
        compiler_params=pltpu.CompilerParams(needs_layout_passes=False),
    )
    return f(xr, ttab, wtab)


@jax.jit
def kernel(x, phi):
    ttab, wtab = _prep_pallas(phi)
    acc, cnt = _sc_push(x, ttab.reshape(B, NCHUNK, CHUNK),
                        wtab.reshape(B, NCHUNK, CHUNK))
    out = _norm_pallas(acc, cnt)
    return out.reshape(B, C, H, W)


# speedup vs baseline: 1.4155x; 1.1694x over previous
import functools

import jax
import jax.numpy as jnp
from jax import lax
from jax.experimental import pallas as pl
from jax.experimental.pallas import tpu as pltpu
from jax.experimental.pallas import tpu_sc as plsc

B = 2
C = 192
H = 224
W = 224
HW = H * W

NUM_CORES = 2
NUM_SUBCORES = 16
LANES = 16

CPT = C // NUM_SUBCORES
PASSES = CPT // 2
CHUNK = 1792
NCHUNK = HW // CHUNK
GROUPS = CHUNK // LANES
NBUF = 4
WSCALE = 1.0 / 65536.0


def _prep_pallas(phi):
    def body(phi_ref, t_ref, w_ref):
        ph0 = phi_ref[0]
        ph1 = phi_ref[1]
        ii = lax.broadcasted_iota(jnp.int32, (H, W), 0).astype(jnp.float32)
        jj = lax.broadcasted_iota(jnp.int32, (H, W), 1).astype(jnp.float32)
        gi = ii + ph0
        gj = jj + ph1
        i0f = jnp.floor(gi)
        j0f = jnp.floor(gj)
        i0 = jnp.mod(i0f.astype(jnp.int32), H)
        j0 = jnp.mod(j0f.astype(jnp.int32), W)
        jwrap = (j0 == W - 1).astype(jnp.int32)
        iwrap = (i0 == H - 1).astype(jnp.int32)
        t_ref[...] = (i0 * W + j0) | (jwrap << 16) | (iwrap << 17)
        wq_i = (jnp.clip(gi - i0f, 0.0, 1.0) * 65536.0).astype(jnp.int32)
        wq_j = (jnp.clip(gj - j0f, 0.0, 1.0) * 65536.0).astype(jnp.int32)
        wq_i = jnp.minimum(wq_i, 65535)
        wq_j = jnp.minimum(wq_j, 65535)
        w_ref[...] = wq_i | (wq_j << 16)

    return pl.pallas_call(
        body,
        grid=(B,),
        in_specs=[pl.BlockSpec((None, 2, H, W), lambda b: (b, 0, 0, 0))],
        out_specs=[
            pl.BlockSpec((None, H, W), lambda b: (b, 0, 0)),
            pl.BlockSpec((None, H, W), lambda b: (b, 0, 0)),
        ],
        out_shape=[
            jax.ShapeDtypeStruct((B, H, W), jnp.int32),
            jax.ShapeDtypeStruct((B, H, W), jnp.int32),
        ],
    )(phi)


def _norm_pallas(acc, cnt):
    CB = 16

    def body(a_ref, c_ref, o_ref):
        total = jnp.sum(c_ref[...], axis=0, keepdims=True)
        o_ref[...] = (a_ref[...] / jnp.maximum(total, 0.001)).reshape(
            CB, H, W)

    return pl.pallas_call(
        body,
        grid=(B, C // CB),
        in_specs=[
            pl.BlockSpec((None, CB, HW), lambda b, c: (b, c, 0)),
            pl.BlockSpec((None, NUM_SUBCORES, HW), lambda b, c: (b, 0, 0)),
        ],
        out_specs=pl.BlockSpec((None, CB, H, W), lambda b, c: (b, c, 0, 0)),
        out_shape=jax.ShapeDtypeStruct((B, C, H, W), jnp.float32),
    )(acc, cnt)


def _decode_group(tbuf, wbuf, s, g):
    base = g * LANES
    tw = tbuf[s, pl.ds(base, LANES)]
    ww = wbuf[s, pl.ds(base, LANES)]
    idx00 = tw & 0xFFFF
    djb = (tw >> 16) & 1
    dib = tw >> 17
    dj = 1 - djb * W
    di = W - dib * HW
    wi = ((ww & 0xFFFF).astype(jnp.float32)) * WSCALE
    wj = (((ww >> 16) & 0xFFFF).astype(jnp.float32)) * WSCALE
    idx01 = idx00 + dj
    idx10 = idx00 + di
    idx11 = idx01 + di
    u = 1.0 - wi
    v = 1.0 - wj
    return (
        (idx00, u * v),
        (idx01, u * wj),
        (idx10, wi * v),
        (idx11, wi * wj),
    )


def _sc_push(xr, ttab, wtab):
    mesh = plsc.VectorSubcoreMesh(core_axis_name="c", subcore_axis_name="s")

    def body(x_hbm, t_hbm, w_hbm, out_hbm, cnt_hbm, acc_a, acc_b, tbuf, wbuf,
             xbuf, *sems):
        cid = lax.axis_index("c")
        sid = lax.axis_index("s")
        sem_t = sems[0:NBUF]
        sem_w = sems[NBUF:2 * NBUF]
        sem_x = sems[2 * NBUF:3 * NBUF]
        sem_oa = sems[3 * NBUF]
        sem_ob = sems[3 * NBUF + 1]

        def tab_src(t):
            return t_hbm.at[cid, t], w_hbm.at[cid, t]

        def start_tab(t, s):
            st, sw = tab_src(t)
            pltpu.async_copy(st, tbuf.at[s], sem_t[s])
            pltpu.async_copy(sw, wbuf.at[s], sem_w[s])

        def wait_tab(t, s):
            st, sw = tab_src(t)
            pltpu.make_async_copy(st, tbuf.at[s], sem_t[s]).wait()
            pltpu.make_async_copy(sw, wbuf.at[s], sem_w[s]).wait()

        def chunk_loop(compute_chunk, extra_start, extra_wait):
            for t0 in range(NBUF - 1):
                start_tab(t0, t0)
                extra_start(t0, t0)

            def quad_body(tq, carry):
                for s in range(NBUF):
                    t = tq * NBUF + s
                    sn = (s + NBUF - 1) % NBUF

                    @pl.when(t + NBUF - 1 < NCHUNK)
                    def _():
                        start_tab(t + NBUF - 1, sn)
                        extra_start(t + NBUF - 1, sn)

                    wait_tab(t, s)
                    extra_wait(t, s)
                    compute_chunk(t, s)

                return carry

            lax.fori_loop(0, NCHUNK // NBUF, quad_body, 0)

        def zero_a():
            @plsc.parallel_loop(0, HW // LANES, 1, unroll=8)
            def _(g):
                acc_a[pl.ds(g * LANES, LANES)] = jnp.zeros((LANES,),
                                                           jnp.float32)

        def zero_ab():
            @plsc.parallel_loop(0, HW // LANES, 1, unroll=8)
            def _(g):
                z = jnp.zeros((LANES,), jnp.float32)
                acc_a[pl.ds(g * LANES, LANES)] = z
                acc_b[pl.ds(g * LANES, LANES)] = z

        zero_a()

        lo = (sid * 7) >> 2
        hi = ((sid + 1) * 7) >> 2

        def cnt_body(t, carry):
            pltpu.sync_copy(t_hbm.at[cid, t], tbuf.at[0])
            pltpu.sync_copy(w_hbm.at[cid, t], wbuf.at[0])

            @plsc.parallel_loop(0, GROUPS, 1, unroll=4)
            def _(g):
                for idx, wt in _decode_group(tbuf, wbuf, 0, g):
                    plsc.addupdate_scatter(acc_a, [idx], wt)

            return carry

        lax.fori_loop(lo, hi, cnt_body, 0)
        pltpu.sync_copy(acc_a, cnt_hbm.at[cid, sid])

        c_base = sid * CPT

        for p in range(PASSES):
            c0 = c_base + 2 * p

            def x_src(t):
                roff = pl.multiple_of(t * (CHUNK // W), 8)
                return x_hbm.at[cid, pl.ds(c0, 2), pl.ds(roff, CHUNK // W), :]

            def start_x(t, s):
                pltpu.async_copy(x_src(t), xbuf.at[s], sem_x[s])

            def wait_x(t, s):
                pltpu.make_async_copy(x_src(t), xbuf.at[s], sem_x[s]).wait()

            if p > 0:
                cprev = c_base + 2 * (p - 1)
                pltpu.make_async_copy(acc_a, out_hbm.at[cid, cprev],
                                      sem_oa).wait()
                pltpu.make_async_copy(acc_b, out_hbm.at[cid, cprev + 1],
                                      sem_ob).wait()
            zero_ab()

            def scat_chunk(t, s):
                del t

                @plsc.parallel_loop(0, GROUPS, 1, unroll=4)
                def _(g):
                    base = g * LANES
                    q = (g * 9363) >> 17
                    col = (g - q * 14) * LANES
                    x0 = xbuf[s, 0, q, pl.ds(col, LANES)]
                    x1 = xbuf[s, 1, q, pl.ds(col, LANES)]
                    for idx, wt in _decode_group(tbuf, wbuf, s, g):
                        plsc.addupdate_scatter(acc_a, [idx], wt * x0)
                        plsc.addupdate_scatter(acc_b, [idx], wt * x1)

            chunk_loop(scat_chunk, start_x, wait_x)

            pltpu.async_copy(acc_a, out_hbm.at[cid, c0], sem_oa)
            pltpu.async_copy(acc_b, out_hbm.at[cid, c0 + 1], sem_ob)

        clast = c_base + 2 * (PASSES - 1)
        pltpu.make_async_copy(acc_a, out_hbm.at[cid, clast], sem_oa).wait()
        pltpu.make_async_copy(acc_b, out_hbm.at[cid, clast + 1],
                              sem_ob).wait()

    f = pl.kernel(
        body,
        out_type=[
            jax.ShapeDtypeStruct((B, C, HW), jnp.float32),
            jax.ShapeDtypeStruct((B, NUM_SUBCORES, HW), jnp.float32),
        ],
        mesh=mesh,
        scratch_types=[
            pltpu.VMEM((HW,), jnp.float32),
            pltpu.VMEM((HW,), jnp.float32),
            pltpu.VMEM((NBUF, CHUNK), jnp.int32),
            pltpu.VMEM((NBUF, CHUNK), jnp.int32),
            pltpu.VMEM((NBUF, 2, CHUNK // W, W), jnp.float32),
        ] + [pltpu.SemaphoreType.DMA] * (3 * NBUF + 2),
        compiler_params=pltpu.CompilerParams(needs_layout_passes=False),
    )
    return f(xr, ttab, wtab)


@jax.jit
def kernel(x, phi):
    ttab, wtab = _prep_pallas(phi)
    acc, cnt = _sc_push(x, ttab.reshape(B, NCHUNK, CHUNK),
                        wtab.reshape(B, NCHUNK, CHUNK))
    return _norm_pallas(acc, cnt)
